# baseline (speedup 1.0000x reference)
import functools

import jax
import jax.numpy as jnp
from jax import lax
from jax.experimental import pallas as pl
from jax.experimental.pallas import tpu as pltpu
from jax.experimental.pallas import tpu_sc as plsc

HID = 64
KH = 50
OMEGA = 30.0

_NC = 2
_NS = 16
_GRP = 128


def _dense_body(rel_ref, emb_ref, w1_ref, b1_ref, g1_ref, be1_ref, wall_ref,
                out_ref):
    x = rel_ref[0]
    h1 = w1_ref[:] * x + b1_ref[:]
    mu = jnp.mean(h1, axis=0, keepdims=True)
    d = h1 - mu
    var = jnp.mean(d * d, axis=0, keepdims=True)
    h = d * jax.lax.rsqrt(var + 1e-5) * g1_ref[:] + be1_ref[:]
    h = jnp.sin(OMEGA * h).astype(jnp.bfloat16)
    embt = emb_ref[:].T.astype(jnp.bfloat16)
    eb = embt.shape[1]
    hexp = jnp.broadcast_to(h[:, None, :], (KH, HID, eb))
    ot = (hexp * embt[None, :, :]).reshape(KH * HID, eb)
    ofull = jnp.concatenate([ot, embt], axis=0)
    msgt = jnp.dot(wall_ref[:], ofull, preferred_element_type=jnp.float32)
    out_ref[:] = msgt.T


def _dense_side(rel3, embg, w1, b1, g1, be1, w2, b2, eb):
    nb = rel3.shape[0]
    e_pad = nb * eb
    kfull = KH * HID + HID
    w2r = w2.reshape(HID, HID, KH)
    wall = jnp.concatenate(
        [w2r.transpose(0, 2, 1).reshape(HID, KH * HID),
         b2.reshape(HID, HID)], axis=1).astype(jnp.bfloat16)
    return pl.pallas_call(
        _dense_body,
        grid=(nb,),
        in_specs=[
            pl.BlockSpec((1, 1, eb), lambda i: (i, 0, 0)),
            pl.BlockSpec((eb, HID), lambda i: (i, 0)),
            pl.BlockSpec((KH, 1), lambda i: (0, 0)),
            pl.BlockSpec((KH, 1), lambda i: (0, 0)),
            pl.BlockSpec((KH, 1), lambda i: (0, 0)),
            pl.BlockSpec((KH, 1), lambda i: (0, 0)),
            pl.BlockSpec((HID, kfull), lambda i: (0, 0)),
        ],
        out_specs=pl.BlockSpec((eb, HID), lambda i: (i, 0)),
        out_shape=jax.ShapeDtypeStruct((e_pad, HID), jnp.float32),
    )(rel3, embg, w1.reshape(KH, 1), b1.reshape(KH, 1), g1.reshape(KH, 1),
      be1.reshape(KH, 1), wall)


def _gather_side(emb, t, idx, et):
    e_pad = idx.shape[0]
    nw = _NC * _NS
    chunk = e_pad // nw
    ngrp = chunk // _GRP
    nn = emb.shape[0]
    mesh = plsc.VectorSubcoreMesh(core_axis_name="c", subcore_axis_name="s")

    @functools.partial(
        pl.kernel, mesh=mesh,
        out_type=(jax.ShapeDtypeStruct((e_pad, HID), jnp.float32),
                  jax.ShapeDtypeStruct((e_pad,), jnp.float32)),
        compiler_params=pltpu.CompilerParams(use_tc_tiling_on_sc=False),
        scratch_types=[
            pltpu.VMEM((chunk,), jnp.int32),
            pltpu.VMEM((ngrp, _GRP), jnp.int32),
            pltpu.VMEM((chunk, HID), jnp.float32),
            pltpu.VMEM((chunk,), jnp.float32),
            pltpu.VMEM((chunk,), jnp.float32),
            pltpu.VMEM((chunk,), jnp.float32),
            pltpu.SemaphoreType.DMA,
            pltpu.SemaphoreType.DMA,
        ],
    )
    def k(emb_h, t_h, idx_h, et_h, eo_h, ro_h,
          idx_v, cl_v, rows_v, tg_v, et_v, rel_v, sem_a, sem_b):
        c = lax.axis_index("c")
        s = lax.axis_index("s")
        base = (c * _NS + s) * chunk
        pltpu.sync_copy(et_h.at[pl.ds(base, chunk)], et_v)
        pltpu.sync_copy(idx_h.at[pl.ds(base, chunk)], idx_v)
        for g in range(ngrp):
            for l in range(_GRP // 16):
                o = g * _GRP + l * 16
                cl_v[g, pl.ds(l * 16, 16)] = jnp.minimum(
                    idx_v[pl.ds(o, 16)], jnp.int32(nn - 1))
        rowc = [pltpu.async_copy(emb_h.at[cl_v.at[g]],
                                 rows_v.at[pl.ds(g * _GRP, _GRP)], sem_a)
                for g in range(ngrp)]
        tc = [pltpu.async_copy(t_h.at[cl_v.at[g]],
                               tg_v.at[pl.ds(g * _GRP, _GRP)], sem_b)
              for g in range(ngrp)]
        for h in tc:
            h.wait()
        for g in range(ngrp):
            for l in range(_GRP // 16):
                o = g * _GRP + l * 16
                rel_v[pl.ds(o, 16)] = tg_v[pl.ds(o, 16)] - et_v[pl.ds(o, 16)]
        pltpu.sync_copy(rel_v, ro_h.at[pl.ds(base, chunk)])
        for h in rowc:
            h.wait()
        pltpu.sync_copy(rows_v, eo_h.at[pl.ds(base, chunk)])

    return k(emb, t, idx, et)


def _scatter_add(msg, idx, zeros, n_rows):
    e_pad = msg.shape[0]
    assert e_pad % (_NS * _GRP) == 0
    chunk = e_pad // _NS
    ngrp = chunk // _GRP
    half = n_rows // 2
    rpt = -(-(-(-half // _NS)) // 8) * 8
    last = half - (_NS - 1) * rpt
    assert last > 0 and last % 8 == 0 and rpt <= zeros.shape[0]
    mesh = plsc.VectorSubcoreMesh(core_axis_name="c", subcore_axis_name="s")

    @functools.partial(
        pl.kernel, mesh=mesh,
        out_type=jax.ShapeDtypeStruct((n_rows, HID), jnp.float32),
        compiler_params=pltpu.CompilerParams(use_tc_tiling_on_sc=False),
        scratch_types=[
            pltpu.VMEM((chunk,), jnp.int32),
            pltpu.VMEM((ngrp, _GRP), jnp.int32),
            pltpu.VMEM((2, _GRP, HID), jnp.float32),
            pltpu.VMEM_SHARED((half + 8, HID), jnp.float32),
            pltpu.SemaphoreType.DMA,
            pltpu.SemaphoreType.DMA,
        ],
    )
    def k(msg_hbm, idx_hbm, zeros_hbm, out_hbm, idx_v, lidx_v, msg_v, acc_sh,
          sem0, sem1):
        c = lax.axis_index("c")
        s = lax.axis_index("s")
        half_i = jnp.int32(half)
        sems = [sem0, sem1]

        @pl.when(s < _NS - 1)
        def _():
            pltpu.sync_copy(zeros_hbm.at[pl.ds(0, rpt)],
                            acc_sh.at[pl.ds(s * rpt, rpt)])

        @pl.when(s == _NS - 1)
        def _():
            pltpu.sync_copy(zeros_hbm.at[pl.ds(0, last)],
                            acc_sh.at[pl.ds(s * rpt, last)])

        base = s * chunk
        pltpu.sync_copy(idx_hbm.at[pl.ds(base, chunk)], idx_v)
        for g in range(ngrp):
            for l in range(_GRP // 16):
                o = g * _GRP + l * 16
                v = idx_v[pl.ds(o, 16)] - c * half_i
                ok = (v >= 0) & (v < half_i)
                lidx_v[g, pl.ds(l * 16, 16)] = jnp.where(ok, v, half_i)

        plsc.subcore_barrier()

        loads = [None, None]
        loads[0] = pltpu.async_copy(
            msg_hbm.at[pl.ds(base, _GRP)], msg_v.at[0], sems[0])
        for g in range(ngrp):
            b = g % 2
            if g + 1 < ngrp:
                loads[1 - b] = pltpu.async_copy(
                    msg_hbm.at[pl.ds(base + (g + 1) * _GRP, _GRP)],
                    msg_v.at[1 - b], sems[1 - b])
            loads[b].wait()
            pltpu.sync_copy(msg_v.at[b], acc_sh.at[lidx_v.at[g]], add=True)

        plsc.subcore_barrier()

        @pl.when(s < _NS - 1)
        def _():
            pltpu.sync_copy(acc_sh.at[pl.ds(s * rpt, rpt)],
                            out_hbm.at[pl.ds(c * half + s * rpt, rpt)])

        @pl.when(s == _NS - 1)
        def _():
            pltpu.sync_copy(acc_sh.at[pl.ds(s * rpt, last)],
                            out_hbm.at[pl.ds(c * half + s * rpt, last)])

    return k(msg, idx, zeros)


def _transpose_out(x, bs):
    n = x.shape[0]
    return pl.pallas_call(
        lambda x_ref, o_ref: o_ref.__setitem__(..., x_ref[:].T),
        grid=((n + bs - 1) // bs,),
        in_specs=[pl.BlockSpec((bs, HID), lambda i: (i, 0))],
        out_specs=pl.BlockSpec((HID, bs), lambda i: (0, i)),
        out_shape=jax.ShapeDtypeStruct((HID, n), jnp.float32),
    )(x)


def kernel(u_embedded, i_embedded, user_per_trans, item_per_trans, edges_t,
           u_t, i_t,
           wu_w1, wu_b1, wu_g1, wu_be1, wu_w2, wu_b2,
           wi_w1, wi_b1, wi_g1, wi_be1, wi_w2, wi_b2):
    e = edges_t.shape[0]
    n_users = u_embedded.shape[0]
    n_items = i_embedded.shape[0]
    eb = 1024
    quantum = _NS * _GRP
    e_pad = ((e + quantum - 1) // quantum) * quantum
    pad = e_pad - e
    nb = e_pad // eb

    up = jnp.pad(user_per_trans, (0, pad), constant_values=n_users)
    ip = jnp.pad(item_per_trans, (0, pad), constant_values=n_items)
    et = jnp.pad(edges_t, (0, pad))

    embg_i, rel_i_f = _gather_side(i_embedded, i_t, ip, et)
    rel_i = rel_i_f.reshape(nb, 1, eb)
    msg_item = _dense_side(rel_i, embg_i, wi_w1, wi_b1, wi_g1, wi_be1, wi_w2,
                           wi_b2, eb)

    embg_u, rel_u_f = _gather_side(u_embedded, u_t, up, et)
    rel_u = rel_u_f.reshape(nb, 1, eb)
    msg_user = _dense_side(rel_u, embg_u, wu_w1, wu_b1, wu_g1, wu_be1, wu_w2,
                           wu_b2, eb)

    zeros = jnp.zeros((-(-max(n_users, n_items) // (2 * _NS * 8)) * 8, HID),
                      jnp.float32)
    hlu = _scatter_add(msg_item, up, zeros, n_users)
    hli = _scatter_add(msg_user, ip, zeros, n_items)
    return (_transpose_out(hlu, 1024).T, _transpose_out(hli, 1024).T)


# speedup vs baseline: 1.1165x; 1.1165x over previous
import functools

import jax
import jax.numpy as jnp
from jax import lax
from jax.experimental import pallas as pl
from jax.experimental.pallas import tpu as pltpu
from jax.experimental.pallas import tpu_sc as plsc

HID = 64
KH = 50
OMEGA = 30.0

_NC = 2
_NS = 16
_GRP = 128


def _dense_body(rel_ref, emb_ref, w1_ref, b1_ref, g1_ref, be1_ref, wall_ref,
                out_ref):
    x = rel_ref[0]
    h1 = w1_ref[:] * x + b1_ref[:]
    mu = jnp.mean(h1, axis=0, keepdims=True)
    d = h1 - mu
    var = jnp.mean(d * d, axis=0, keepdims=True)
    h = d * jax.lax.rsqrt(var + 1e-5) * g1_ref[:] + be1_ref[:]
    h = jnp.sin(OMEGA * h).astype(jnp.bfloat16)
    embt = emb_ref[:].T.astype(jnp.bfloat16)
    eb = embt.shape[1]
    hexp = jnp.broadcast_to(h[:, None, :], (KH, HID, eb))
    ot = (hexp * embt[None, :, :]).reshape(KH * HID, eb)
    ofull = jnp.concatenate([ot, embt], axis=0)
    msgt = jnp.dot(wall_ref[:], ofull, preferred_element_type=jnp.float32)
    out_ref[:] = msgt.T


def _dense_side(rel3, embg, w1, b1, g1, be1, w2, b2, eb):
    nb = rel3.shape[0]
    e_pad = nb * eb
    kfull = KH * HID + HID
    w2r = w2.reshape(HID, HID, KH)
    wall = jnp.concatenate(
        [w2r.transpose(0, 2, 1).reshape(HID, KH * HID),
         b2.reshape(HID, HID)], axis=1).astype(jnp.bfloat16)
    return pl.pallas_call(
        _dense_body,
        grid=(nb,),
        in_specs=[
            pl.BlockSpec((1, 1, eb), lambda i: (i, 0, 0)),
            pl.BlockSpec((eb, HID), lambda i: (i, 0)),
            pl.BlockSpec((KH, 1), lambda i: (0, 0)),
            pl.BlockSpec((KH, 1), lambda i: (0, 0)),
            pl.BlockSpec((KH, 1), lambda i: (0, 0)),
            pl.BlockSpec((KH, 1), lambda i: (0, 0)),
            pl.BlockSpec((HID, kfull), lambda i: (0, 0)),
        ],
        out_specs=pl.BlockSpec((eb, HID), lambda i: (i, 0)),
        out_shape=jax.ShapeDtypeStruct((e_pad, HID), jnp.float32),
    )(rel3, embg, w1.reshape(KH, 1), b1.reshape(KH, 1), g1.reshape(KH, 1),
      be1.reshape(KH, 1), wall)


def _gather_side(emb, t, idx, et):
    e_pad = idx.shape[0]
    nw = _NC * _NS
    chunk = e_pad // nw
    ngrp = chunk // _GRP
    nn = emb.shape[0]
    mesh = plsc.VectorSubcoreMesh(core_axis_name="c", subcore_axis_name="s")

    @functools.partial(
        pl.kernel, mesh=mesh,
        out_type=(jax.ShapeDtypeStruct((e_pad, HID), jnp.float32),
                  jax.ShapeDtypeStruct((e_pad,), jnp.float32)),
        compiler_params=pltpu.CompilerParams(use_tc_tiling_on_sc=False),
        scratch_types=[
            pltpu.VMEM((chunk,), jnp.int32),
            pltpu.VMEM((ngrp, _GRP), jnp.int32),
            pltpu.VMEM((chunk, HID), jnp.float32),
            pltpu.VMEM((chunk,), jnp.float32),
            pltpu.VMEM((chunk,), jnp.float32),
            pltpu.VMEM((chunk,), jnp.float32),
            pltpu.SemaphoreType.DMA,
            pltpu.SemaphoreType.DMA,
        ],
    )
    def k(emb_h, t_h, idx_h, et_h, eo_h, ro_h,
          idx_v, cl_v, rows_v, tg_v, et_v, rel_v, sem_a, sem_b):
        c = lax.axis_index("c")
        s = lax.axis_index("s")
        base = (c * _NS + s) * chunk
        pltpu.sync_copy(et_h.at[pl.ds(base, chunk)], et_v)
        pltpu.sync_copy(idx_h.at[pl.ds(base, chunk)], idx_v)
        for g in range(ngrp):
            for l in range(_GRP // 16):
                o = g * _GRP + l * 16
                cl_v[g, pl.ds(l * 16, 16)] = jnp.minimum(
                    idx_v[pl.ds(o, 16)], jnp.int32(nn - 1))
        rowc = [pltpu.async_copy(emb_h.at[cl_v.at[g]],
                                 rows_v.at[pl.ds(g * _GRP, _GRP)], sem_a)
                for g in range(ngrp)]
        tc = [pltpu.async_copy(t_h.at[cl_v.at[g]],
                               tg_v.at[pl.ds(g * _GRP, _GRP)], sem_b)
              for g in range(ngrp)]
        for h in tc:
            h.wait()
        for g in range(ngrp):
            for l in range(_GRP // 16):
                o = g * _GRP + l * 16
                rel_v[pl.ds(o, 16)] = tg_v[pl.ds(o, 16)] - et_v[pl.ds(o, 16)]
        pltpu.sync_copy(rel_v, ro_h.at[pl.ds(base, chunk)])
        for h in rowc:
            h.wait()
        pltpu.sync_copy(rows_v, eo_h.at[pl.ds(base, chunk)])

    return k(emb, t, idx, et)


def _scatter_add(msg, idx, zeros, n_rows):
    e_pad = msg.shape[0]
    assert e_pad % (_NS * _GRP) == 0
    chunk = e_pad // _NS
    ngrp = chunk // _GRP
    half = n_rows // 2
    rpt = -(-(-(-half // _NS)) // 8) * 8
    last = half - (_NS - 1) * rpt
    assert last > 0 and last % 8 == 0 and rpt <= zeros.shape[0]
    mesh = plsc.VectorSubcoreMesh(core_axis_name="c", subcore_axis_name="s")

    @functools.partial(
        pl.kernel, mesh=mesh,
        out_type=jax.ShapeDtypeStruct((n_rows, HID), jnp.float32),
        compiler_params=pltpu.CompilerParams(use_tc_tiling_on_sc=False),
        scratch_types=[
            pltpu.VMEM((chunk,), jnp.int32),
            pltpu.VMEM((ngrp, _GRP), jnp.int32),
            pltpu.VMEM((2, _GRP, HID), jnp.float32),
            pltpu.VMEM_SHARED((half + 8, HID), jnp.float32),
            pltpu.SemaphoreType.DMA,
            pltpu.SemaphoreType.DMA,
        ],
    )
    def k(msg_hbm, idx_hbm, zeros_hbm, out_hbm, idx_v, lidx_v, msg_v, acc_sh,
          sem0, sem1):
        c = lax.axis_index("c")
        s = lax.axis_index("s")
        half_i = jnp.int32(half)
        sems = [sem0, sem1]

        @pl.when(s < _NS - 1)
        def _():
            pltpu.sync_copy(zeros_hbm.at[pl.ds(0, rpt)],
                            acc_sh.at[pl.ds(s * rpt, rpt)])

        @pl.when(s == _NS - 1)
        def _():
            pltpu.sync_copy(zeros_hbm.at[pl.ds(0, last)],
                            acc_sh.at[pl.ds(s * rpt, last)])

        base = s * chunk
        pltpu.sync_copy(idx_hbm.at[pl.ds(base, chunk)], idx_v)
        for g in range(ngrp):
            for l in range(_GRP // 16):
                o = g * _GRP + l * 16
                v = idx_v[pl.ds(o, 16)] - c * half_i
                ok = (v >= 0) & (v < half_i)
                lidx_v[g, pl.ds(l * 16, 16)] = jnp.where(ok, v, half_i)

        plsc.subcore_barrier()

        loads = [None, None]
        loads[0] = pltpu.async_copy(
            msg_hbm.at[pl.ds(base, _GRP)], msg_v.at[0], sems[0])
        for g in range(ngrp):
            b = g % 2
            if g + 1 < ngrp:
                loads[1 - b] = pltpu.async_copy(
                    msg_hbm.at[pl.ds(base + (g + 1) * _GRP, _GRP)],
                    msg_v.at[1 - b], sems[1 - b])
            loads[b].wait()
            pltpu.sync_copy(msg_v.at[b], acc_sh.at[lidx_v.at[g]], add=True)

        plsc.subcore_barrier()

        @pl.when(s < _NS - 1)
        def _():
            pltpu.sync_copy(acc_sh.at[pl.ds(s * rpt, rpt)],
                            out_hbm.at[pl.ds(c * half + s * rpt, rpt)])

        @pl.when(s == _NS - 1)
        def _():
            pltpu.sync_copy(acc_sh.at[pl.ds(s * rpt, last)],
                            out_hbm.at[pl.ds(c * half + s * rpt, last)])

    return k(msg, idx, zeros)


def _transpose_out(x, bs):
    n = x.shape[0]
    return pl.pallas_call(
        lambda x_ref, o_ref: o_ref.__setitem__(..., x_ref[:].T),
        grid=((n + bs - 1) // bs,),
        in_specs=[pl.BlockSpec((bs, HID), lambda i: (i, 0))],
        out_specs=pl.BlockSpec((HID, bs), lambda i: (0, i)),
        out_shape=jax.ShapeDtypeStruct((HID, n), jnp.float32),
    )(x)


def kernel(u_embedded, i_embedded, user_per_trans, item_per_trans, edges_t,
           u_t, i_t,
           wu_w1, wu_b1, wu_g1, wu_be1, wu_w2, wu_b2,
           wi_w1, wi_b1, wi_g1, wi_be1, wi_w2, wi_b2):
    e = edges_t.shape[0]
    n_users = u_embedded.shape[0]
    n_items = i_embedded.shape[0]
    eb = 1024
    quantum = _NS * _GRP
    e_pad = ((e + quantum - 1) // quantum) * quantum
    pad = e_pad - e
    nb = e_pad // eb

    up = jnp.pad(user_per_trans, (0, pad), constant_values=n_users)
    ip = jnp.pad(item_per_trans, (0, pad), constant_values=n_items)
    et = jnp.pad(edges_t, (0, pad))

    embg_i, rel_i_f = _gather_side(i_embedded, i_t, ip, et)
    rel_i = rel_i_f.reshape(nb, 1, eb)
    msg_item = _dense_side(rel_i, embg_i, wi_w1, wi_b1, wi_g1, wi_be1, wi_w2,
                           wi_b2, eb)

    embg_u, rel_u_f = _gather_side(u_embedded, u_t, up, et)
    rel_u = rel_u_f.reshape(nb, 1, eb)
    msg_user = _dense_side(rel_u, embg_u, wu_w1, wu_b1, wu_g1, wu_be1, wu_w2,
                           wu_b2, eb)

    zeros = jnp.zeros((-(-max(n_users, n_items) // (2 * _NS * 8)) * 8, HID),
                      jnp.float32)
    hlu = _scatter_add(msg_item, up, zeros, n_users)
    hli = _scatter_add(msg_user, ip, zeros, n_items)
    return (hlu, hli)


# speedup vs baseline: 1.1283x; 1.0105x over previous
import functools

import jax
import jax.numpy as jnp
from jax import lax
from jax.experimental import pallas as pl
from jax.experimental.pallas import tpu as pltpu
from jax.experimental.pallas import tpu_sc as plsc

HID = 64
KH = 50
OMEGA = 30.0

_NC = 2
_NS = 16
_GRP = 128


def _dense_body(rel_ref, emb_ref, w1_ref, b1_ref, g1_ref, be1_ref, wall_ref,
                out_ref):
    x = rel_ref[0]
    h1 = w1_ref[:] * x + b1_ref[:]
    mu = jnp.mean(h1, axis=0, keepdims=True)
    d = h1 - mu
    var = jnp.mean(d * d, axis=0, keepdims=True)
    h = d * jax.lax.rsqrt(var + 1e-5) * g1_ref[:] + be1_ref[:]
    h = jnp.sin(OMEGA * h).astype(jnp.bfloat16)
    embt = emb_ref[:].T.astype(jnp.bfloat16)
    eb = embt.shape[1]
    hexp = jnp.broadcast_to(h[:, None, :], (KH, HID, eb))
    ot = (hexp * embt[None, :, :]).reshape(KH * HID, eb)
    ofull = jnp.concatenate([ot, embt], axis=0)
    msgt = jnp.dot(wall_ref[:], ofull, preferred_element_type=jnp.float32)
    out_ref[:] = msgt.T


def _dense_side(rel3, embg, w1, b1, g1, be1, w2, b2, eb):
    nb = rel3.shape[0]
    e_pad = nb * eb
    kfull = KH * HID + HID
    w2r = w2.reshape(HID, HID, KH)
    wall = jnp.concatenate(
        [w2r.transpose(0, 2, 1).reshape(HID, KH * HID),
         b2.reshape(HID, HID)], axis=1).astype(jnp.bfloat16)
    return pl.pallas_call(
        _dense_body,
        grid=(nb,),
        in_specs=[
            pl.BlockSpec((1, 1, eb), lambda i: (i, 0, 0)),
            pl.BlockSpec((eb, HID), lambda i: (i, 0)),
            pl.BlockSpec((KH, 1), lambda i: (0, 0)),
            pl.BlockSpec((KH, 1), lambda i: (0, 0)),
            pl.BlockSpec((KH, 1), lambda i: (0, 0)),
            pl.BlockSpec((KH, 1), lambda i: (0, 0)),
            pl.BlockSpec((HID, kfull), lambda i: (0, 0)),
        ],
        out_specs=pl.BlockSpec((eb, HID), lambda i: (i, 0)),
        out_shape=jax.ShapeDtypeStruct((e_pad, HID), jnp.float32),
    )(rel3, embg, w1.reshape(KH, 1), b1.reshape(KH, 1), g1.reshape(KH, 1),
      be1.reshape(KH, 1), wall)


def _gather_side(emb, t, idx, et):
    e_pad = idx.shape[0]
    nw = _NC * _NS
    chunk = e_pad // nw
    ngrp = chunk // _GRP
    nn = emb.shape[0]
    mesh = plsc.VectorSubcoreMesh(core_axis_name="c", subcore_axis_name="s")

    @functools.partial(
        pl.kernel, mesh=mesh,
        out_type=(jax.ShapeDtypeStruct((e_pad, HID), jnp.float32),
                  jax.ShapeDtypeStruct((e_pad,), jnp.float32)),
        compiler_params=pltpu.CompilerParams(use_tc_tiling_on_sc=False,
                                             skip_device_barrier=True),
        scratch_types=[
            pltpu.VMEM((chunk,), jnp.int32),
            pltpu.VMEM((ngrp, _GRP), jnp.int32),
            pltpu.VMEM((chunk, HID), jnp.float32),
            pltpu.VMEM((chunk,), jnp.float32),
            pltpu.VMEM((chunk,), jnp.float32),
            pltpu.VMEM((chunk,), jnp.float32),
            pltpu.SemaphoreType.DMA,
            pltpu.SemaphoreType.DMA,
        ],
    )
    def k(emb_h, t_h, idx_h, et_h, eo_h, ro_h,
          idx_v, cl_v, rows_v, tg_v, et_v, rel_v, sem_a, sem_b):
        c = lax.axis_index("c")
        s = lax.axis_index("s")
        base = (c * _NS + s) * chunk
        h_et = pltpu.async_copy(et_h.at[pl.ds(base, chunk)], et_v, sem_a)
        h_ix = pltpu.async_copy(idx_h.at[pl.ds(base, chunk)], idx_v, sem_b)
        h_ix.wait()
        for g in range(ngrp):
            for l in range(_GRP // 16):
                o = g * _GRP + l * 16
                cl_v[g, pl.ds(l * 16, 16)] = jnp.minimum(
                    idx_v[pl.ds(o, 16)], jnp.int32(nn - 1))
        h_et.wait()
        rowc = [pltpu.async_copy(emb_h.at[cl_v.at[g]],
                                 rows_v.at[pl.ds(g * _GRP, _GRP)], sem_a)
                for g in range(ngrp)]
        tc = [pltpu.async_copy(t_h.at[cl_v.at[g]],
                               tg_v.at[pl.ds(g * _GRP, _GRP)], sem_b)
              for g in range(ngrp)]
        for h in tc:
            h.wait()
        for g in range(ngrp):
            for l in range(_GRP // 16):
                o = g * _GRP + l * 16
                rel_v[pl.ds(o, 16)] = tg_v[pl.ds(o, 16)] - et_v[pl.ds(o, 16)]
        pltpu.sync_copy(rel_v, ro_h.at[pl.ds(base, chunk)])
        for h in rowc:
            h.wait()
        pltpu.sync_copy(rows_v, eo_h.at[pl.ds(base, chunk)])

    return k(emb, t, idx, et)


def _scatter_add(msg, idx, zeros, n_rows):
    e_pad = msg.shape[0]
    assert e_pad % (_NS * _GRP) == 0
    chunk = e_pad // _NS
    ngrp = chunk // _GRP
    half = n_rows // 2
    rpt = -(-(-(-half // _NS)) // 8) * 8
    last = half - (_NS - 1) * rpt
    assert last > 0 and last % 8 == 0 and rpt <= zeros.shape[0]
    mesh = plsc.VectorSubcoreMesh(core_axis_name="c", subcore_axis_name="s")

    @functools.partial(
        pl.kernel, mesh=mesh,
        out_type=jax.ShapeDtypeStruct((n_rows, HID), jnp.float32),
        compiler_params=pltpu.CompilerParams(use_tc_tiling_on_sc=False,
                                             skip_device_barrier=True),
        scratch_types=[
            pltpu.VMEM((chunk,), jnp.int32),
            pltpu.VMEM((ngrp, _GRP), jnp.int32),
            pltpu.VMEM((2, _GRP, HID), jnp.float32),
            pltpu.VMEM_SHARED((half + 16, HID), jnp.float32),
            pltpu.SemaphoreType.DMA,
            pltpu.SemaphoreType.DMA,
        ],
    )
    def k(msg_hbm, idx_hbm, zeros_hbm, out_hbm, idx_v, lidx_v, msg_v, acc_sh,
          sem0, sem1):
        c = lax.axis_index("c")
        s = lax.axis_index("s")
        half_i = jnp.int32(half)
        dump = jnp.int32(half) + lax.iota(jnp.int32, 16)
        sems = [sem0, sem1]

        @pl.when(s < _NS - 1)
        def _():
            pltpu.sync_copy(zeros_hbm.at[pl.ds(0, rpt)],
                            acc_sh.at[pl.ds(s * rpt, rpt)])

        @pl.when(s == _NS - 1)
        def _():
            pltpu.sync_copy(zeros_hbm.at[pl.ds(0, last)],
                            acc_sh.at[pl.ds(s * rpt, last)])

        base = s * chunk
        pltpu.sync_copy(idx_hbm.at[pl.ds(base, chunk)], idx_v)
        for g in range(ngrp):
            for l in range(_GRP // 16):
                o = g * _GRP + l * 16
                v = idx_v[pl.ds(o, 16)] - c * half_i
                ok = (v >= 0) & (v < half_i)
                lidx_v[g, pl.ds(l * 16, 16)] = jnp.where(ok, v, dump)

        plsc.subcore_barrier()

        loads = [None, None]
        loads[0] = pltpu.async_copy(
            msg_hbm.at[pl.ds(base, _GRP)], msg_v.at[0], sems[0])
        for g in range(ngrp):
            b = g % 2
            if g + 1 < ngrp:
                loads[1 - b] = pltpu.async_copy(
                    msg_hbm.at[pl.ds(base + (g + 1) * _GRP, _GRP)],
                    msg_v.at[1 - b], sems[1 - b])
            loads[b].wait()
            pltpu.sync_copy(msg_v.at[b], acc_sh.at[lidx_v.at[g]], add=True)

        plsc.subcore_barrier()

        @pl.when(s < _NS - 1)
        def _():
            pltpu.sync_copy(acc_sh.at[pl.ds(s * rpt, rpt)],
                            out_hbm.at[pl.ds(c * half + s * rpt, rpt)])

        @pl.when(s == _NS - 1)
        def _():
            pltpu.sync_copy(acc_sh.at[pl.ds(s * rpt, last)],
                            out_hbm.at[pl.ds(c * half + s * rpt, last)])

    return k(msg, idx, zeros)


def _transpose_out(x, bs):
    n = x.shape[0]
    return pl.pallas_call(
        lambda x_ref, o_ref: o_ref.__setitem__(..., x_ref[:].T),
        grid=((n + bs - 1) // bs,),
        in_specs=[pl.BlockSpec((bs, HID), lambda i: (i, 0))],
        out_specs=pl.BlockSpec((HID, bs), lambda i: (0, i)),
        out_shape=jax.ShapeDtypeStruct((HID, n), jnp.float32),
    )(x)


def kernel(u_embedded, i_embedded, user_per_trans, item_per_trans, edges_t,
           u_t, i_t,
           wu_w1, wu_b1, wu_g1, wu_be1, wu_w2, wu_b2,
           wi_w1, wi_b1, wi_g1, wi_be1, wi_w2, wi_b2):
    e = edges_t.shape[0]
    n_users = u_embedded.shape[0]
    n_items = i_embedded.shape[0]
    eb = 1024
    quantum = _NS * _GRP
    e_pad = ((e + quantum - 1) // quantum) * quantum
    pad = e_pad - e
    nb = e_pad // eb

    up = jnp.pad(user_per_trans, (0, pad), constant_values=n_users)
    ip = jnp.pad(item_per_trans, (0, pad), constant_values=n_items)
    et = jnp.pad(edges_t, (0, pad))

    embg_i, rel_i_f = _gather_side(i_embedded, i_t, ip, et)
    rel_i = rel_i_f.reshape(nb, 1, eb)
    msg_item = _dense_side(rel_i, embg_i, wi_w1, wi_b1, wi_g1, wi_be1, wi_w2,
                           wi_b2, eb)

    embg_u, rel_u_f = _gather_side(u_embedded, u_t, up, et)
    rel_u = rel_u_f.reshape(nb, 1, eb)
    msg_user = _dense_side(rel_u, embg_u, wu_w1, wu_b1, wu_g1, wu_be1, wu_w2,
                           wu_b2, eb)

    zeros = jnp.zeros((-(-max(n_users, n_items) // (2 * _NS * 8)) * 8, HID),
                      jnp.float32)
    hlu = _scatter_add(msg_item, up, zeros, n_users)
    hli = _scatter_add(msg_user, ip, zeros, n_items)
    return (hlu, hli)


# speedup vs baseline: 1.1284x; 1.0001x over previous
import functools

import jax
import jax.numpy as jnp
from jax import lax
from jax.experimental import pallas as pl
from jax.experimental.pallas import tpu as pltpu
from jax.experimental.pallas import tpu_sc as plsc

HID = 64
KH = 50
OMEGA = 30.0

_NC = 2
_NS = 16
_GRP = 128


def _dense_body(rel_ref, emb_ref, w1_ref, b1_ref, g1_ref, be1_ref, wall_ref,
                out_ref):
    x = rel_ref[0]
    h1 = w1_ref[:] * x + b1_ref[:]
    mu = jnp.mean(h1, axis=0, keepdims=True)
    d = h1 - mu
    var = jnp.mean(d * d, axis=0, keepdims=True)
    h = d * jax.lax.rsqrt(var + 1e-5) * g1_ref[:] + be1_ref[:]
    h = jnp.sin(OMEGA * h).astype(jnp.bfloat16)
    embt = emb_ref[:].T.astype(jnp.bfloat16)
    eb = embt.shape[1]
    hexp = jnp.broadcast_to(h[:, None, :], (KH, HID, eb))
    ot = (hexp * embt[None, :, :]).reshape(KH * HID, eb)
    ofull = jnp.concatenate([ot, embt], axis=0)
    msgt = jnp.dot(wall_ref[:], ofull, preferred_element_type=jnp.float32)
    out_ref[:] = msgt.T


def _dense_side(rel3, embg, w1, b1, g1, be1, w2, b2, eb):
    nb = rel3.shape[0]
    e_pad = nb * eb
    kfull = KH * HID + HID
    w2r = w2.reshape(HID, HID, KH)
    wall = jnp.concatenate(
        [w2r.transpose(0, 2, 1).reshape(HID, KH * HID),
         b2.reshape(HID, HID)], axis=1).astype(jnp.bfloat16)
    return pl.pallas_call(
        _dense_body,
        grid=(nb,),
        in_specs=[
            pl.BlockSpec((1, 1, eb), lambda i: (i, 0, 0)),
            pl.BlockSpec((eb, HID), lambda i: (i, 0)),
            pl.BlockSpec((KH, 1), lambda i: (0, 0)),
            pl.BlockSpec((KH, 1), lambda i: (0, 0)),
            pl.BlockSpec((KH, 1), lambda i: (0, 0)),
            pl.BlockSpec((KH, 1), lambda i: (0, 0)),
            pl.BlockSpec((HID, kfull), lambda i: (0, 0)),
        ],
        out_specs=pl.BlockSpec((eb, HID), lambda i: (i, 0)),
        out_shape=jax.ShapeDtypeStruct((e_pad, HID), jnp.float32),
    )(rel3, embg, w1.reshape(KH, 1), b1.reshape(KH, 1), g1.reshape(KH, 1),
      be1.reshape(KH, 1), wall)


def _gather_side(emb, t, idx, et):
    e_pad = idx.shape[0]
    nw = _NC * _NS
    chunk = e_pad // nw
    ngrp = chunk // _GRP
    nn = emb.shape[0]
    mesh = plsc.VectorSubcoreMesh(core_axis_name="c", subcore_axis_name="s")

    @functools.partial(
        pl.kernel, mesh=mesh,
        out_type=(jax.ShapeDtypeStruct((e_pad, HID), jnp.float32),
                  jax.ShapeDtypeStruct((e_pad,), jnp.float32)),
        compiler_params=pltpu.CompilerParams(use_tc_tiling_on_sc=False,
                                             skip_device_barrier=True),
        scratch_types=[
            pltpu.VMEM((chunk,), jnp.int32),
            pltpu.VMEM((ngrp, _GRP), jnp.int32),
            pltpu.VMEM((chunk, HID), jnp.float32),
            pltpu.VMEM((chunk,), jnp.float32),
            pltpu.VMEM((chunk,), jnp.float32),
            pltpu.VMEM((chunk,), jnp.float32),
            pltpu.SemaphoreType.DMA,
            pltpu.SemaphoreType.DMA,
        ],
    )
    def k(emb_h, t_h, idx_h, et_h, eo_h, ro_h,
          idx_v, cl_v, rows_v, tg_v, et_v, rel_v, sem_a, sem_b):
        c = lax.axis_index("c")
        s = lax.axis_index("s")
        base = (c * _NS + s) * chunk
        h_et = pltpu.async_copy(et_h.at[pl.ds(base, chunk)], et_v, sem_a)
        h_ix = pltpu.async_copy(idx_h.at[pl.ds(base, chunk)], idx_v, sem_b)
        h_ix.wait()
        for g in range(ngrp):
            for l in range(_GRP // 16):
                o = g * _GRP + l * 16
                cl_v[g, pl.ds(l * 16, 16)] = jnp.minimum(
                    idx_v[pl.ds(o, 16)], jnp.int32(nn - 1))
        h_et.wait()
        rowc = [pltpu.async_copy(emb_h.at[cl_v.at[g]],
                                 rows_v.at[pl.ds(g * _GRP, _GRP)], sem_a)
                for g in range(ngrp)]
        tc = [pltpu.async_copy(t_h.at[cl_v.at[g]],
                               tg_v.at[pl.ds(g * _GRP, _GRP)], sem_b)
              for g in range(ngrp)]
        for h in tc:
            h.wait()
        for g in range(ngrp):
            for l in range(_GRP // 16):
                o = g * _GRP + l * 16
                rel_v[pl.ds(o, 16)] = tg_v[pl.ds(o, 16)] - et_v[pl.ds(o, 16)]
        pltpu.sync_copy(rel_v, ro_h.at[pl.ds(base, chunk)])
        for h in rowc:
            h.wait()
        pltpu.sync_copy(rows_v, eo_h.at[pl.ds(base, chunk)])

    return k(emb, t, idx, et)


def _scatter_add(msg, idx, zeros, n_rows):
    e_pad = msg.shape[0]
    assert e_pad % (_NS * _GRP) == 0
    chunk = e_pad // _NS
    ngrp = chunk // _GRP
    half = n_rows // 2
    rpt = -(-(-(-half // _NS)) // 8) * 8
    last = half - (_NS - 1) * rpt
    assert last > 0 and last % 8 == 0 and rpt <= zeros.shape[0]
    mesh = plsc.VectorSubcoreMesh(core_axis_name="c", subcore_axis_name="s")

    @functools.partial(
        pl.kernel, mesh=mesh,
        out_type=jax.ShapeDtypeStruct((n_rows, HID), jnp.float32),
        compiler_params=pltpu.CompilerParams(use_tc_tiling_on_sc=False,
                                             skip_device_barrier=True),
        scratch_types=[
            pltpu.VMEM((chunk,), jnp.int32),
            pltpu.VMEM((ngrp, _GRP), jnp.int32),
            pltpu.VMEM((2, _GRP, HID), jnp.float32),
            pltpu.VMEM_SHARED((half + 16, HID), jnp.float32),
            pltpu.SemaphoreType.DMA,
            pltpu.SemaphoreType.DMA,
            pltpu.SemaphoreType.DMA,
            pltpu.SemaphoreType.DMA,
        ],
    )
    def k(msg_hbm, idx_hbm, zeros_hbm, out_hbm, idx_v, lidx_v, msg_v, acc_sh,
          sem0, sem1, sem2, sem3):
        c = lax.axis_index("c")
        s = lax.axis_index("s")
        half_i = jnp.int32(half)
        dump = jnp.int32(half) + lax.iota(jnp.int32, 16)
        sems = [sem0, sem1]
        scat_sems = [sem2, sem3]

        @pl.when(s < _NS - 1)
        def _():
            pltpu.sync_copy(zeros_hbm.at[pl.ds(0, rpt)],
                            acc_sh.at[pl.ds(s * rpt, rpt)])

        @pl.when(s == _NS - 1)
        def _():
            pltpu.sync_copy(zeros_hbm.at[pl.ds(0, last)],
                            acc_sh.at[pl.ds(s * rpt, last)])

        base = s * chunk
        pltpu.sync_copy(idx_hbm.at[pl.ds(base, chunk)], idx_v)
        for g in range(ngrp):
            for l in range(_GRP // 16):
                o = g * _GRP + l * 16
                v = idx_v[pl.ds(o, 16)] - c * half_i
                ok = (v >= 0) & (v < half_i)
                lidx_v[g, pl.ds(l * 16, 16)] = jnp.where(ok, v, dump)

        plsc.subcore_barrier()

        loads = [None, None]
        scats = [None, None]
        loads[0] = pltpu.async_copy(
            msg_hbm.at[pl.ds(base, _GRP)], msg_v.at[0], sems[0])
        for g in range(ngrp):
            b = g % 2
            if g + 1 < ngrp:
                if scats[1 - b] is not None:
                    scats[1 - b].wait()
                loads[1 - b] = pltpu.async_copy(
                    msg_hbm.at[pl.ds(base + (g + 1) * _GRP, _GRP)],
                    msg_v.at[1 - b], sems[1 - b])
            loads[b].wait()
            scats[b] = pltpu.async_copy(
                msg_v.at[b], acc_sh.at[lidx_v.at[g]], scat_sems[b], add=True)
        for b in range(2):
            if scats[b] is not None:
                scats[b].wait()

        plsc.subcore_barrier()

        @pl.when(s < _NS - 1)
        def _():
            pltpu.sync_copy(acc_sh.at[pl.ds(s * rpt, rpt)],
                            out_hbm.at[pl.ds(c * half + s * rpt, rpt)])

        @pl.when(s == _NS - 1)
        def _():
            pltpu.sync_copy(acc_sh.at[pl.ds(s * rpt, last)],
                            out_hbm.at[pl.ds(c * half + s * rpt, last)])

    return k(msg, idx, zeros)


def _transpose_out(x, bs):
    n = x.shape[0]
    return pl.pallas_call(
        lambda x_ref, o_ref: o_ref.__setitem__(..., x_ref[:].T),
        grid=((n + bs - 1) // bs,),
        in_specs=[pl.BlockSpec((bs, HID), lambda i: (i, 0))],
        out_specs=pl.BlockSpec((HID, bs), lambda i: (0, i)),
        out_shape=jax.ShapeDtypeStruct((HID, n), jnp.float32),
    )(x)


def kernel(u_embedded, i_embedded, user_per_trans, item_per_trans, edges_t,
           u_t, i_t,
           wu_w1, wu_b1, wu_g1, wu_be1, wu_w2, wu_b2,
           wi_w1, wi_b1, wi_g1, wi_be1, wi_w2, wi_b2):
    e = edges_t.shape[0]
    n_users = u_embedded.shape[0]
    n_items = i_embedded.shape[0]
    eb = 1024
    quantum = _NS * _GRP
    e_pad = ((e + quantum - 1) // quantum) * quantum
    pad = e_pad - e
    nb = e_pad // eb

    up = jnp.pad(user_per_trans, (0, pad), constant_values=n_users)
    ip = jnp.pad(item_per_trans, (0, pad), constant_values=n_items)
    et = jnp.pad(edges_t, (0, pad))

    embg_i, rel_i_f = _gather_side(i_embedded, i_t, ip, et)
    rel_i = rel_i_f.reshape(nb, 1, eb)
    msg_item = _dense_side(rel_i, embg_i, wi_w1, wi_b1, wi_g1, wi_be1, wi_w2,
                           wi_b2, eb)

    embg_u, rel_u_f = _gather_side(u_embedded, u_t, up, et)
    rel_u = rel_u_f.reshape(nb, 1, eb)
    msg_user = _dense_side(rel_u, embg_u, wu_w1, wu_b1, wu_g1, wu_be1, wu_w2,
                           wu_b2, eb)

    zeros = jnp.zeros((-(-max(n_users, n_items) // (2 * _NS * 8)) * 8, HID),
                      jnp.float32)
    hlu = _scatter_add(msg_item, up, zeros, n_users)
    hli = _scatter_add(msg_user, ip, zeros, n_items)
    return (hlu, hli)
